# baseline (speedup 1.0000x reference)
import functools

import jax
import jax.numpy as jnp
from jax import lax
from jax.experimental import pallas as pl
from jax.experimental.pallas import tpu as pltpu
from jax.experimental.pallas import tpu_sc as plsc

_MAX_COUNT = 255
_D = 128
_NC = 2
_NS = 16
_NW = _NC * _NS


@functools.lru_cache(maxsize=None)
def _make_lookup(total):
    per_w = total // _NW
    R = 320
    steps = per_w // R
    assert per_w % R == 0 and steps % 2 == 0

    mesh = plsc.VectorSubcoreMesh(core_axis_name="c", subcore_axis_name="s")

    @functools.partial(
        pl.kernel,
        mesh=mesh,
        out_type=jax.ShapeDtypeStruct((total, _D), jnp.float32),
        scratch_types=[
            pltpu.VMEM((_MAX_COUNT + 1, _D), jnp.float32),
            pltpu.VMEM((R,), jnp.int32),
            pltpu.VMEM((R,), jnp.int32),
            pltpu.VMEM((R, _D), jnp.float32),
            pltpu.VMEM((R, _D), jnp.float32),
            pltpu.SemaphoreType.DMA,
            pltpu.SemaphoreType.DMA,
            pltpu.SemaphoreType.DMA,
            pltpu.SemaphoreType.DMA,
        ],
    )
    def lookup(idx_hbm, tbl_hbm, ext_hbm, out_hbm,
               tbl_v, idx_v0, idx_v1, rows_v0, rows_v1,
               isem0, isem1, wsem0, wsem1):
        wid = lax.axis_index("s") * _NC + lax.axis_index("c")
        base = wid * per_w
        idx_v = (idx_v0, idx_v1)
        rows_v = (rows_v0, rows_v1)
        isem = (isem0, isem1)
        wsem = (wsem0, wsem1)

        pltpu.sync_copy(tbl_hbm, tbl_v)
        pltpu.sync_copy(ext_hbm, tbl_v.at[pl.ds(_MAX_COUNT, 1)])

        def fire_i(s, b):
            pltpu.async_copy(idx_hbm.at[pl.ds(base + s * R, R)],
                             idx_v[b], isem[b])

        def wait_i(b):
            pltpu.make_async_copy(idx_hbm.at[pl.ds(base, R)],
                                  idx_v[b], isem[b]).wait()

        def fire_w(s, b):
            pltpu.async_copy(rows_v[b],
                             out_hbm.at[pl.ds(base + s * R, R)],
                             wsem[b])

        def wait_w(b):
            pltpu.make_async_copy(rows_v[b],
                                  out_hbm.at[pl.ds(base, R)],
                                  wsem[b]).wait()

        def compute(b):
            ib = idx_v[b]
            rb = rows_v[b]

            def group(g, carry):
                jv = ib[pl.ds(g * 16, 16)]
                for l in range(16):
                    j = jv[l]
                    r = g * 16 + l
                    for t in range(_D // 16):
                        rb[r, pl.ds(16 * t, 16)] = tbl_v[j, pl.ds(16 * t, 16)]
                return carry

            lax.fori_loop(0, R // 16, group, 0)

        fire_i(0, 0)

        def pair(p, carry):
            wait_i(0)
            fire_i(2 * p + 1, 1)

            @pl.when(p >= 1)
            def _():
                wait_w(0)

            compute(0)
            fire_w(2 * p, 0)

            wait_i(1)

            @pl.when(p + 1 < steps // 2)
            def _():
                fire_i(2 * p + 2, 0)

            @pl.when(p >= 1)
            def _():
                wait_w(1)

            compute(1)
            fire_w(2 * p + 1, 1)
            return carry

        lax.fori_loop(0, steps // 2, pair, 0)
        wait_w(0)
        wait_w(1)

    return lookup


def kernel(tensor, int_to_feat_matrix, extra_embeddings):
    batch, fields = tensor.shape
    total = batch * fields

    idx = tensor.astype(jnp.int32).reshape(total)
    out2d = _make_lookup(total)(idx, int_to_feat_matrix, extra_embeddings)
    return out2d.reshape(batch, fields * _D)


# speedup vs baseline: 2.0137x; 2.0137x over previous
import functools

import jax
import jax.numpy as jnp
from jax import lax
from jax.experimental import pallas as pl
from jax.experimental.pallas import tpu as pltpu
from jax.experimental.pallas import tpu_sc as plsc

_MAX_COUNT = 255
_D = 128
_NC = 2
_NS = 16
_NW = _NC * _NS


@functools.lru_cache(maxsize=None)
def _make_lookup(nrows2d):
    rows_per_w = nrows2d // _NW
    K = 2
    steps = rows_per_w // K
    R = K * 128
    assert rows_per_w % K == 0 and steps % 2 == 0

    mesh = plsc.VectorSubcoreMesh(core_axis_name="c", subcore_axis_name="s")

    @functools.partial(
        pl.kernel,
        mesh=mesh,
        out_type=jax.ShapeDtypeStruct((nrows2d * _D, _D), jnp.float32),
        scratch_types=[
            pltpu.VMEM_SHARED((_MAX_COUNT + 1, _D), jnp.float32),
            pltpu.VMEM((K, 128), jnp.int32),
            pltpu.VMEM((K, 128), jnp.int32),
            pltpu.VMEM((R, _D), jnp.float32),
            pltpu.VMEM((R, _D), jnp.float32),
            pltpu.SemaphoreType.DMA,
            pltpu.SemaphoreType.DMA,
            pltpu.SemaphoreType.DMA,
            pltpu.SemaphoreType.DMA,
            pltpu.SemaphoreType.DMA,
            pltpu.SemaphoreType.DMA,
        ],
    )
    def lookup(idx_hbm, tbl_hbm, ext_hbm, out_hbm,
               tbl_sh, idx_v0, idx_v1, rows_v0, rows_v1,
               isem0, isem1, gsem0, gsem1, wsem0, wsem1):
        cid = lax.axis_index("c")
        sid = lax.axis_index("s")
        wid = sid * _NC + cid
        row0 = wid * rows_per_w
        idx_v = (idx_v0, idx_v1)
        rows_v = (rows_v0, rows_v1)
        isem = (isem0, isem1)
        gsem = (gsem0, gsem1)
        wsem = (wsem0, wsem1)

        @pl.when(sid == 0)
        def _():
            pltpu.sync_copy(tbl_hbm, rows_v0)
            pltpu.sync_copy(ext_hbm, rows_v0.at[pl.ds(_MAX_COUNT, 1)])
            pltpu.sync_copy(rows_v0, tbl_sh)

        plsc.subcore_barrier()

        def fire_i(s, b):
            pltpu.async_copy(idx_hbm.at[pl.ds(row0 + s * K, K)],
                             idx_v[b], isem[b])

        def wait_i(b):
            pltpu.make_async_copy(idx_hbm.at[pl.ds(row0, K)],
                                  idx_v[b], isem[b]).wait()

        def fire_g(b):
            for j in range(K):
                pltpu.async_copy(tbl_sh.at[idx_v[b].at[j]],
                                 rows_v[b].at[pl.ds(j * 128, 128)],
                                 gsem[b])

        def wait_g(b):
            for j in range(K):
                pltpu.make_async_copy(tbl_sh.at[idx_v[b].at[j]],
                                      rows_v[b].at[pl.ds(j * 128, 128)],
                                      gsem[b]).wait()

        def fire_w(s, b):
            pltpu.async_copy(rows_v[b],
                             out_hbm.at[pl.ds((row0 + s * K) * 128, R)],
                             wsem[b])

        def wait_w(b):
            pltpu.make_async_copy(rows_v[b],
                                  out_hbm.at[pl.ds(row0 * 128, R)],
                                  wsem[b]).wait()

        fire_i(0, 0)

        def pair(p, carry):
            wait_i(0)

            @pl.when(p >= 1)
            def _():
                wait_w(0)

            fire_g(0)

            @pl.when(p >= 1)
            def _():
                wait_g(1)
                fire_w(2 * p - 1, 1)

            fire_i(2 * p + 1, 1)

            wait_i(1)

            @pl.when(p >= 1)
            def _():
                wait_w(1)

            fire_g(1)
            wait_g(0)
            fire_w(2 * p, 0)

            @pl.when(p + 1 < steps // 2)
            def _():
                fire_i(2 * p + 2, 0)
            return carry

        lax.fori_loop(0, steps // 2, pair, 0)

        wait_g(1)
        fire_w(steps - 1, 1)
        wait_w(0)
        wait_w(1)

    return lookup


def kernel(tensor, int_to_feat_matrix, extra_embeddings):
    batch, fields = tensor.shape
    total = batch * fields
    nrows2d = total // 128
    assert total % 128 == 0

    idx2d = tensor.astype(jnp.int32).reshape(nrows2d, 128)
    out2d = _make_lookup(nrows2d)(idx2d, int_to_feat_matrix, extra_embeddings)
    return out2d.reshape(batch, fields * _D)
